# baseline (speedup 1.0000x reference)
import functools

import jax
import jax.numpy as jnp
from jax import lax
from jax.experimental import pallas as pl
from jax.experimental.pallas import tpu as pltpu
from jax.experimental.pallas import tpu_sc as plsc

_TM = 256


def _router_body(x_ref, wg_ref, bg_ref, wn_ref, bn_ref, eps_ref,
                 i0_ref, i1_ref, r0_ref, r1_ref, p0_ref, p1_ref, cnt_ref,
                 carry_ref, tri_ref):
    @pl.when(pl.program_id(0) == 0)
    def _():
        carry_ref[...] = jnp.zeros_like(carry_ref)
        tm = tri_ref.shape[0]
        ri = lax.broadcasted_iota(jnp.int32, (tm, tm), 0)
        ci = lax.broadcasted_iota(jnp.int32, (tm, tm), 1)
        tri_ref[...] = jnp.where(ri > ci, 1.0, 0.0)

    x = x_ref[...]
    logits = jnp.dot(x, wg_ref[...]) + bg_ref[...]
    nz = jnp.dot(x, wn_ref[...]) + bn_ref[...]
    sp = jnp.maximum(nz, 0.0) + jnp.log1p(jnp.exp(-jnp.abs(nz)))
    noisy = logits + eps_ref[...] * sp

    e_dim = noisy.shape[1]
    ii = lax.broadcasted_iota(jnp.int32, noisy.shape, 1)
    m0 = jnp.max(noisy, axis=1, keepdims=True)
    i0 = jnp.min(jnp.where(noisy == m0, ii, e_dim), axis=1, keepdims=True)
    n1 = jnp.where(ii == i0, -jnp.inf, noisy)
    m1 = jnp.max(n1, axis=1, keepdims=True)
    i1 = jnp.min(jnp.where(n1 == m1, ii, e_dim), axis=1, keepdims=True)
    sel = (ii == i0) | (ii == i1)
    pu = jnp.where(sel, jnp.exp(noisy - m0), 0.0)
    denom = jnp.sum(pu, axis=1, keepdims=True)
    i0_ref[...] = i0
    i1_ref[...] = i1
    p0_ref[...] = 1.0 / denom
    p1_ref[...] = jnp.exp(m1 - m0) / denom

    oh = ((ii == i0).astype(jnp.float32) + (ii == i1).astype(jnp.float32))
    excl = jnp.dot(tri_ref[...], oh, precision=lax.Precision.HIGHEST)
    carry = carry_ref[...]
    tot = excl + carry
    r0_ref[...] = jnp.sum(jnp.where(ii == i0, tot, 0.0), axis=1,
                          keepdims=True).astype(jnp.int32)
    r1_ref[...] = jnp.sum(jnp.where(ii == i1, tot, 0.0), axis=1,
                          keepdims=True).astype(jnp.int32)
    new_cnt = carry + jnp.sum(oh, axis=0, keepdims=True)
    carry_ref[...] = new_cnt
    cnt_ref[...] = new_cnt


def _router(x2, Wg, bg, Wn, bn, eps, tm):
    t, d = x2.shape
    e = Wg.shape[1]
    col = lambda i: (i, 0)
    rep = lambda i: (0, 0)
    outs = (
        [jax.ShapeDtypeStruct((t, 1), jnp.int32)] * 2
        + [jax.ShapeDtypeStruct((t, 1), jnp.int32)] * 2
        + [jax.ShapeDtypeStruct((t, 1), jnp.float32)] * 2
        + [jax.ShapeDtypeStruct((1, e), jnp.float32)]
    )
    return pl.pallas_call(
        _router_body,
        grid=(t // tm,),
        in_specs=[
            pl.BlockSpec((tm, d), col),
            pl.BlockSpec((d, e), rep),
            pl.BlockSpec((1, e), rep),
            pl.BlockSpec((d, e), rep),
            pl.BlockSpec((1, e), rep),
            pl.BlockSpec((tm, e), col),
        ],
        out_specs=[pl.BlockSpec((tm, 1), col)] * 6 + [pl.BlockSpec((1, e), rep)],
        out_shape=outs,
        scratch_shapes=[pltpu.VMEM((1, e), jnp.float32),
                        pltpu.VMEM((tm, tm), jnp.float32)],
        compiler_params=pltpu.CompilerParams(
            dimension_semantics=("arbitrary",)),
    )(x2, Wg, bg.reshape(1, e), Wn, bn.reshape(1, e), eps)


def _pos_body(cnt_ref, i0_ref, i1_ref, r0_ref, r1_ref,
              pos0_ref, pos1_ref, emap_ref, *, tm_g, n_tiles):
    cnt = cnt_ref[...]
    e = cnt.shape[1]
    padc = jnp.ceil(cnt / tm_g) * tm_g
    ri = lax.broadcasted_iota(jnp.int32, (e, e), 0)
    ci = lax.broadcasted_iota(jnp.int32, (e, e), 1)
    m = jnp.where(ri < ci, 1.0, 0.0)
    starts = jnp.dot(padc, m, precision=lax.Precision.HIGHEST)

    t = i0_ref.shape[0]
    ii = lax.broadcasted_iota(jnp.int32, (t, e), 1)
    st_b = jnp.broadcast_to(starts, (t, e))
    st0 = jnp.sum(jnp.where(ii == i0_ref[...], st_b, 0.0), axis=1,
                  keepdims=True).astype(jnp.int32)
    st1 = jnp.sum(jnp.where(ii == i1_ref[...], st_b, 0.0), axis=1,
                  keepdims=True).astype(jnp.int32)
    pos0_ref[...] = st0 + r0_ref[...]
    pos1_ref[...] = st1 + r1_ref[...]

    ends = starts + padc
    tv = (lax.broadcasted_iota(jnp.int32, (n_tiles, 1), 0) * tm_g
          ).astype(jnp.float32)
    ends_b = jnp.broadcast_to(ends, (n_tiles, e))
    ge = (jnp.broadcast_to(tv, (n_tiles, e)) >= ends_b).astype(jnp.int32)
    emap_ref[...] = jnp.minimum(jnp.sum(ge, axis=1, keepdims=True), e - 1)


def _pos(cnt, i0, i1, r0, r1, tm_g, n_tiles):
    t = i0.shape[0]
    e = cnt.shape[1]
    body = functools.partial(_pos_body, tm_g=tm_g, n_tiles=n_tiles)
    return pl.pallas_call(
        body,
        out_shape=[
            jax.ShapeDtypeStruct((t, 1), jnp.int32),
            jax.ShapeDtypeStruct((t, 1), jnp.int32),
            jax.ShapeDtypeStruct((n_tiles, 1), jnp.int32),
        ],
    )(cnt, i0, i1, r0, r1)


def _dispatch_sc(x2, pos0, pos1, n_pad):
    t, dw = x2.shape
    nw = 32
    tpw = t // nw
    c = 32
    nb = 2
    mesh = plsc.VectorSubcoreMesh(core_axis_name="c", subcore_axis_name="s")

    @functools.partial(
        pl.kernel, mesh=mesh,
        out_type=jax.ShapeDtypeStruct((n_pad, dw), jnp.float32),
        scratch_types=[
            *([pltpu.VMEM((c, dw), jnp.float32)] * nb),
            *([pltpu.VMEM((c,), jnp.int32)] * (2 * nb)),
            *([pltpu.SemaphoreType.DMA] * (2 * nb)),
        ],
    )
    def disp(x_hbm, p0_hbm, p1_hbm, xs_hbm,
             rows0, rows1, ia0, ia1, ib0, ib1,
             si0, si1, so0, so1):
        rows = (rows0, rows1)
        idx0 = (ia0, ib0)
        idx1 = (ia1, ib1)
        sin = (si0, si1)
        sout = (so0, so1)
        wid = lax.axis_index("s") * 2 + lax.axis_index("c")
        pend = {}
        for ci in range(tpw // c):
            b = ci % nb
            if ci >= nb:
                for h in pend[b]:
                    h.wait()
            base = wid * tpw + ci * c
            h1 = pltpu.async_copy(x_hbm.at[pl.ds(base, c)], rows[b], sin[b])
            h2 = pltpu.async_copy(p0_hbm.at[pl.ds(base, c)], idx0[b], sin[b])
            h3 = pltpu.async_copy(p1_hbm.at[pl.ds(base, c)], idx1[b], sin[b])
            h1.wait()
            h2.wait()
            h3.wait()
            o1 = pltpu.async_copy(rows[b], xs_hbm.at[idx0[b]], sout[b])
            o2 = pltpu.async_copy(rows[b], xs_hbm.at[idx1[b]], sout[b])
            pend[b] = (o1, o2)
        for b in range(nb):
            for h in pend[b]:
                h.wait()

    return disp(x2, pos0, pos1)


def _ffn_body(emap_ref, xs_ref, w1_ref, b1_ref, w2_ref, b2_ref, ys_ref):
    e = emap_ref[pl.program_id(0)]
    xb = xs_ref[...].astype(jnp.bfloat16)
    h = jnp.dot(xb, w1_ref[0], preferred_element_type=jnp.float32)
    b1 = b1_ref[...]
    er1 = lax.broadcasted_iota(jnp.int32, b1.shape, 0)
    h = h + jnp.sum(jnp.where(er1 == e, b1, 0.0), axis=0, keepdims=True)
    h = jnp.maximum(h, 0.0).astype(jnp.bfloat16)
    y = jnp.dot(h, w2_ref[0], preferred_element_type=jnp.float32)
    b2 = b2_ref[...]
    er2 = lax.broadcasted_iota(jnp.int32, b2.shape, 0)
    ys_ref[...] = y + jnp.sum(jnp.where(er2 == e, b2, 0.0), axis=0,
                              keepdims=True)


def _ffn(xs, W1b, b1, W2b, b2, emap, tm):
    n_pad, d = xs.shape
    e, _, h = W1b.shape
    n_tiles = n_pad // tm
    grid_spec = pltpu.PrefetchScalarGridSpec(
        num_scalar_prefetch=1,
        grid=(n_tiles,),
        in_specs=[
            pl.BlockSpec((tm, d), lambda i, em: (i, 0)),
            pl.BlockSpec((1, d, h), lambda i, em: (em[i], 0, 0)),
            pl.BlockSpec((e, h), lambda i, em: (0, 0)),
            pl.BlockSpec((1, h, d), lambda i, em: (em[i], 0, 0)),
            pl.BlockSpec((e, d), lambda i, em: (0, 0)),
        ],
        out_specs=pl.BlockSpec((tm, d), lambda i, em: (i, 0)),
    )
    return pl.pallas_call(
        _ffn_body,
        grid_spec=grid_spec,
        out_shape=jax.ShapeDtypeStruct((n_pad, d), jnp.float32),
        compiler_params=pltpu.CompilerParams(
            dimension_semantics=("arbitrary",)),
    )(emap, xs, W1b, b1, W2b, b2)


def _gather_sc(ys, pos0, pos1, t):
    n_pad, dw = ys.shape
    nw = 32
    tpw = t // nw
    c = 16
    nb = 2
    mesh = plsc.VectorSubcoreMesh(core_axis_name="c", subcore_axis_name="s")

    @functools.partial(
        pl.kernel, mesh=mesh,
        out_type=[jax.ShapeDtypeStruct((t, dw), jnp.float32),
                  jax.ShapeDtypeStruct((t, dw), jnp.float32)],
        scratch_types=[
            *([pltpu.VMEM((c, dw), jnp.float32)] * (2 * nb)),
            *([pltpu.VMEM((c,), jnp.int32)] * (2 * nb)),
            *([pltpu.SemaphoreType.DMA] * (2 * nb)),
        ],
    )
    def gat(ys_hbm, p0_hbm, p1_hbm, y0_hbm, y1_hbm,
            r0a, r0b, r1a, r1b, ia0, ia1, ib0, ib1,
            sa0, sa1, sb0, sb1):
        rows0 = (r0a, r0b)
        rows1 = (r1a, r1b)
        idx0 = (ia0, ib0)
        idx1 = (ia1, ib1)
        sg = (sa0, sb0)
        ss = (sa1, sb1)
        wid = lax.axis_index("s") * 2 + lax.axis_index("c")
        pend = {}
        for ci in range(tpw // c):
            b = ci % nb
            if ci >= nb:
                for h in pend[b]:
                    h.wait()
            base = wid * tpw + ci * c
            h1 = pltpu.async_copy(p0_hbm.at[pl.ds(base, c)], idx0[b], sg[b])
            h2 = pltpu.async_copy(p1_hbm.at[pl.ds(base, c)], idx1[b], sg[b])
            h1.wait()
            h2.wait()
            g0 = pltpu.async_copy(ys_hbm.at[idx0[b]], rows0[b], sg[b])
            g1 = pltpu.async_copy(ys_hbm.at[idx1[b]], rows1[b], sg[b])
            g0.wait()
            g1.wait()
            s0 = pltpu.async_copy(rows0[b], y0_hbm.at[pl.ds(base, c)], ss[b])
            s1 = pltpu.async_copy(rows1[b], y1_hbm.at[pl.ds(base, c)], ss[b])
            pend[b] = (s0, s1)
        for b in range(nb):
            for h in pend[b]:
                h.wait()

    return gat(ys, pos0, pos1)


def _combine_body(y0_ref, y1_ref, p0_ref, p1_ref, out_ref):
    out_ref[...] = y0_ref[...] * p0_ref[...] + y1_ref[...] * p1_ref[...]


def _combine(y0, y1, p0, p1, tm):
    t, d = y0.shape
    col = lambda i: (i, 0)
    return pl.pallas_call(
        _combine_body,
        grid=(t // tm,),
        in_specs=[
            pl.BlockSpec((tm, d), col),
            pl.BlockSpec((tm, d), col),
            pl.BlockSpec((tm, 1), col),
            pl.BlockSpec((tm, 1), col),
        ],
        out_specs=pl.BlockSpec((tm, d), col),
        out_shape=jax.ShapeDtypeStruct((t, d), jnp.float32),
    )(y0, y1, p0, p1)


def _moe_slice(x2, eps, Wg, bg, Wn, bn, W1b, b1, W2b, b2):
    t, d = x2.shape
    e = Wg.shape[-1]
    tmr = 512 if t % 512 == 0 else t
    i0, i1, r0, r1, p0, p1, cnt = _router(x2, Wg, bg, Wn, bn, eps, tmr)

    tm = _TM if t % _TM == 0 else t
    n_pad = 2 * t + e * tm
    n_tiles = n_pad // tm
    pos0, pos1, emap = _pos(cnt, i0, i1, r0, r1, tm, n_tiles)
    pos0f, pos1f = pos0.reshape(t), pos1.reshape(t)

    xs = _dispatch_sc(x2, pos0f, pos1f, n_pad)
    ys = _ffn(xs, W1b, b1, W2b, b2, emap.reshape(n_tiles), tm)
    y0, y1 = _gather_sc(ys, pos0f, pos1f, t)
    return _combine(y0, y1, p0, p1, 1024 if t % 1024 == 0 else t)


def kernel(x, Wg, bg, Wn, bn, W1, b1, W2, b2):
    b, s, d = x.shape
    e = Wg.shape[-1]
    t = b * s
    x2 = x.reshape(t, d)
    eps = jax.random.normal(jax.random.key(42), (b, s, e),
                            dtype=jnp.float32).reshape(t, e)
    W1b, W2b = W1.astype(jnp.bfloat16), W2.astype(jnp.bfloat16)

    if t % 8192 == 0:
        h = t // 2
        outs = [
            _moe_slice(x2[o:o + h], eps[o:o + h], Wg, bg, Wn, bn,
                       W1b, b1, W2b, b2)
            for o in (0, h)
        ]
        out2 = jnp.concatenate(outs, axis=0)
    else:
        out2 = _moe_slice(x2, eps, Wg, bg, Wn, bn, W1b, b1, W2b, b2)
    return out2.reshape(b, s, d)


# speedup vs baseline: 1.1469x; 1.1469x over previous
import functools

import jax
import jax.numpy as jnp
from jax import lax
from jax.experimental import pallas as pl
from jax.experimental.pallas import tpu as pltpu
from jax.experimental.pallas import tpu_sc as plsc

_TM = 256


def _router_body(x_ref, wg_ref, bg_ref, wn_ref, bn_ref, eps_ref,
                 i0_ref, i1_ref, r0_ref, r1_ref, p0_ref, p1_ref, cnt_ref,
                 carry_ref, tri_ref):
    @pl.when(pl.program_id(0) == 0)
    def _():
        carry_ref[...] = jnp.zeros_like(carry_ref)
        tm = tri_ref.shape[0]
        ri = lax.broadcasted_iota(jnp.int32, (tm, tm), 0)
        ci = lax.broadcasted_iota(jnp.int32, (tm, tm), 1)
        tri_ref[...] = jnp.where(ri > ci, 1.0, 0.0)

    x = x_ref[...]
    logits = jnp.dot(x, wg_ref[...]) + bg_ref[...]
    nz = jnp.dot(x, wn_ref[...]) + bn_ref[...]
    sp = jnp.maximum(nz, 0.0) + jnp.log1p(jnp.exp(-jnp.abs(nz)))
    noisy = logits + eps_ref[...] * sp

    e_dim = noisy.shape[1]
    ii = lax.broadcasted_iota(jnp.int32, noisy.shape, 1)
    m0 = jnp.max(noisy, axis=1, keepdims=True)
    i0 = jnp.min(jnp.where(noisy == m0, ii, e_dim), axis=1, keepdims=True)
    n1 = jnp.where(ii == i0, -jnp.inf, noisy)
    m1 = jnp.max(n1, axis=1, keepdims=True)
    i1 = jnp.min(jnp.where(n1 == m1, ii, e_dim), axis=1, keepdims=True)
    sel = (ii == i0) | (ii == i1)
    pu = jnp.where(sel, jnp.exp(noisy - m0), 0.0)
    denom = jnp.sum(pu, axis=1, keepdims=True)
    i0_ref[...] = i0
    i1_ref[...] = i1
    p0_ref[...] = 1.0 / denom
    p1_ref[...] = jnp.exp(m1 - m0) / denom

    oh = ((ii == i0).astype(jnp.float32) + (ii == i1).astype(jnp.float32))
    excl = jnp.dot(tri_ref[...], oh, precision=lax.Precision.HIGHEST)
    carry = carry_ref[...]
    tot = excl + carry
    r0_ref[...] = jnp.sum(jnp.where(ii == i0, tot, 0.0), axis=1,
                          keepdims=True).astype(jnp.int32)
    r1_ref[...] = jnp.sum(jnp.where(ii == i1, tot, 0.0), axis=1,
                          keepdims=True).astype(jnp.int32)
    new_cnt = carry + jnp.sum(oh, axis=0, keepdims=True)
    carry_ref[...] = new_cnt
    cnt_ref[...] = new_cnt


def _router(x2, Wg, bg, Wn, bn, eps, tm):
    t, d = x2.shape
    e = Wg.shape[1]
    col = lambda i: (i, 0)
    rep = lambda i: (0, 0)
    outs = (
        [jax.ShapeDtypeStruct((t, 1), jnp.int32)] * 2
        + [jax.ShapeDtypeStruct((t, 1), jnp.int32)] * 2
        + [jax.ShapeDtypeStruct((t, 1), jnp.float32)] * 2
        + [jax.ShapeDtypeStruct((1, e), jnp.float32)]
    )
    return pl.pallas_call(
        _router_body,
        grid=(t // tm,),
        in_specs=[
            pl.BlockSpec((tm, d), col),
            pl.BlockSpec((d, e), rep),
            pl.BlockSpec((1, e), rep),
            pl.BlockSpec((d, e), rep),
            pl.BlockSpec((1, e), rep),
            pl.BlockSpec((tm, e), col),
        ],
        out_specs=[pl.BlockSpec((tm, 1), col)] * 6 + [pl.BlockSpec((1, e), rep)],
        out_shape=outs,
        scratch_shapes=[pltpu.VMEM((1, e), jnp.float32),
                        pltpu.VMEM((tm, tm), jnp.float32)],
        compiler_params=pltpu.CompilerParams(
            dimension_semantics=("arbitrary",)),
    )(x2, Wg, bg.reshape(1, e), Wn, bn.reshape(1, e), eps)


def _pos_body(cnt_ref, i0_ref, i1_ref, r0_ref, r1_ref,
              pos0_ref, pos1_ref, emap_ref, *, tm_g, n_tiles):
    cnt = cnt_ref[...]
    e = cnt.shape[1]
    padc = jnp.ceil(cnt / tm_g) * tm_g
    ri = lax.broadcasted_iota(jnp.int32, (e, e), 0)
    ci = lax.broadcasted_iota(jnp.int32, (e, e), 1)
    m = jnp.where(ri < ci, 1.0, 0.0)
    starts = jnp.dot(padc, m, precision=lax.Precision.HIGHEST)

    t = i0_ref.shape[0]
    ii = lax.broadcasted_iota(jnp.int32, (t, e), 1)
    st_b = jnp.broadcast_to(starts, (t, e))
    st0 = jnp.sum(jnp.where(ii == i0_ref[...], st_b, 0.0), axis=1,
                  keepdims=True).astype(jnp.int32)
    st1 = jnp.sum(jnp.where(ii == i1_ref[...], st_b, 0.0), axis=1,
                  keepdims=True).astype(jnp.int32)
    pos0_ref[...] = st0 + r0_ref[...]
    pos1_ref[...] = st1 + r1_ref[...]

    ends = starts + padc
    tv = (lax.broadcasted_iota(jnp.int32, (n_tiles, 1), 0) * tm_g
          ).astype(jnp.float32)
    ends_b = jnp.broadcast_to(ends, (n_tiles, e))
    ge = (jnp.broadcast_to(tv, (n_tiles, e)) >= ends_b).astype(jnp.int32)
    emap_ref[...] = jnp.minimum(jnp.sum(ge, axis=1, keepdims=True), e - 1)


def _pos(cnt, i0, i1, r0, r1, tm_g, n_tiles):
    t = i0.shape[0]
    e = cnt.shape[1]
    body = functools.partial(_pos_body, tm_g=tm_g, n_tiles=n_tiles)
    return pl.pallas_call(
        body,
        out_shape=[
            jax.ShapeDtypeStruct((t, 1), jnp.int32),
            jax.ShapeDtypeStruct((t, 1), jnp.int32),
            jax.ShapeDtypeStruct((n_tiles, 1), jnp.int32),
        ],
    )(cnt, i0, i1, r0, r1)


def _dispatch_sc(x2, pos0, pos1, n_pad):
    t, dw = x2.shape
    nw = 32
    tpw = t // nw
    c = 32
    nb = 2
    mesh = plsc.VectorSubcoreMesh(core_axis_name="c", subcore_axis_name="s")

    @functools.partial(
        pl.kernel, mesh=mesh,
        out_type=jax.ShapeDtypeStruct((n_pad, dw), jnp.float32),
        scratch_types=[
            *([pltpu.VMEM((c, dw), jnp.float32)] * nb),
            *([pltpu.VMEM((c,), jnp.int32)] * (2 * nb)),
            *([pltpu.SemaphoreType.DMA] * (2 * nb)),
        ],
    )
    def disp(x_hbm, p0_hbm, p1_hbm, xs_hbm,
             rows0, rows1, ia0, ia1, ib0, ib1,
             si0, si1, so0, so1):
        rows = (rows0, rows1)
        idx0 = (ia0, ib0)
        idx1 = (ia1, ib1)
        sin = (si0, si1)
        sout = (so0, so1)
        wid = lax.axis_index("s") * 2 + lax.axis_index("c")
        pend = {}
        for ci in range(tpw // c):
            b = ci % nb
            if ci >= nb:
                for h in pend[b]:
                    h.wait()
            base = wid * tpw + ci * c
            h1 = pltpu.async_copy(x_hbm.at[pl.ds(base, c)], rows[b], sin[b])
            h2 = pltpu.async_copy(p0_hbm.at[pl.ds(base, c)], idx0[b], sin[b])
            h3 = pltpu.async_copy(p1_hbm.at[pl.ds(base, c)], idx1[b], sin[b])
            h1.wait()
            h2.wait()
            h3.wait()
            o1 = pltpu.async_copy(rows[b], xs_hbm.at[idx0[b]], sout[b])
            o2 = pltpu.async_copy(rows[b], xs_hbm.at[idx1[b]], sout[b])
            pend[b] = (o1, o2)
        for b in range(nb):
            for h in pend[b]:
                h.wait()

    return disp(x2, pos0, pos1)


def _ffn_body(emap_ref, xs_ref, w1_ref, b1_ref, w2_ref, b2_ref, ys_ref):
    e = emap_ref[pl.program_id(0)]
    xb = xs_ref[...].astype(jnp.bfloat16)
    h = jnp.dot(xb, w1_ref[0], preferred_element_type=jnp.float32)
    b1 = b1_ref[...]
    er1 = lax.broadcasted_iota(jnp.int32, b1.shape, 0)
    h = h + jnp.sum(jnp.where(er1 == e, b1, 0.0), axis=0, keepdims=True)
    h = jnp.maximum(h, 0.0).astype(jnp.bfloat16)
    y = jnp.dot(h, w2_ref[0], preferred_element_type=jnp.float32)
    b2 = b2_ref[...]
    er2 = lax.broadcasted_iota(jnp.int32, b2.shape, 0)
    ys_ref[...] = y + jnp.sum(jnp.where(er2 == e, b2, 0.0), axis=0,
                              keepdims=True)


def _ffn(xs, W1b, b1, W2b, b2, emap, tm):
    n_pad, d = xs.shape
    e, _, h = W1b.shape
    n_tiles = n_pad // tm
    grid_spec = pltpu.PrefetchScalarGridSpec(
        num_scalar_prefetch=1,
        grid=(n_tiles,),
        in_specs=[
            pl.BlockSpec((tm, d), lambda i, em: (i, 0)),
            pl.BlockSpec((1, d, h), lambda i, em: (em[i], 0, 0)),
            pl.BlockSpec((e, h), lambda i, em: (0, 0)),
            pl.BlockSpec((1, h, d), lambda i, em: (em[i], 0, 0)),
            pl.BlockSpec((e, d), lambda i, em: (0, 0)),
        ],
        out_specs=pl.BlockSpec((tm, d), lambda i, em: (i, 0)),
    )
    return pl.pallas_call(
        _ffn_body,
        grid_spec=grid_spec,
        out_shape=jax.ShapeDtypeStruct((n_pad, d), jnp.float32),
        compiler_params=pltpu.CompilerParams(
            dimension_semantics=("arbitrary",)),
    )(emap, xs, W1b, b1, W2b, b2)


def _gather_sc(ys, pos0, pos1, t):
    n_pad, dw = ys.shape
    nw = 32
    tpw = t // nw
    c = 16
    nb = 2
    mesh = plsc.VectorSubcoreMesh(core_axis_name="c", subcore_axis_name="s")

    @functools.partial(
        pl.kernel, mesh=mesh,
        out_type=[jax.ShapeDtypeStruct((t, dw), jnp.float32),
                  jax.ShapeDtypeStruct((t, dw), jnp.float32)],
        scratch_types=[
            *([pltpu.VMEM((c, dw), jnp.float32)] * (2 * nb)),
            *([pltpu.VMEM((c,), jnp.int32)] * (2 * nb)),
            *([pltpu.SemaphoreType.DMA] * (2 * nb)),
        ],
    )
    def gat(ys_hbm, p0_hbm, p1_hbm, y0_hbm, y1_hbm,
            r0a, r0b, r1a, r1b, ia0, ia1, ib0, ib1,
            sa0, sa1, sb0, sb1):
        rows0 = (r0a, r0b)
        rows1 = (r1a, r1b)
        idx0 = (ia0, ib0)
        idx1 = (ia1, ib1)
        sg = (sa0, sb0)
        ss = (sa1, sb1)
        wid = lax.axis_index("s") * 2 + lax.axis_index("c")
        pend = {}
        for ci in range(tpw // c):
            b = ci % nb
            if ci >= nb:
                for h in pend[b]:
                    h.wait()
            base = wid * tpw + ci * c
            h1 = pltpu.async_copy(p0_hbm.at[pl.ds(base, c)], idx0[b], sg[b])
            h2 = pltpu.async_copy(p1_hbm.at[pl.ds(base, c)], idx1[b], sg[b])
            h1.wait()
            h2.wait()
            g0 = pltpu.async_copy(ys_hbm.at[idx0[b]], rows0[b], sg[b])
            g1 = pltpu.async_copy(ys_hbm.at[idx1[b]], rows1[b], sg[b])
            g0.wait()
            g1.wait()
            s0 = pltpu.async_copy(rows0[b], y0_hbm.at[pl.ds(base, c)], ss[b])
            s1 = pltpu.async_copy(rows1[b], y1_hbm.at[pl.ds(base, c)], ss[b])
            pend[b] = (s0, s1)
        for b in range(nb):
            for h in pend[b]:
                h.wait()

    return gat(ys, pos0, pos1)


def _combine_body(y0_ref, y1_ref, p0_ref, p1_ref, out_ref):
    out_ref[...] = y0_ref[...] * p0_ref[...] + y1_ref[...] * p1_ref[...]


def _combine(y0, y1, p0, p1, tm):
    t, d = y0.shape
    col = lambda i: (i, 0)
    return pl.pallas_call(
        _combine_body,
        grid=(t // tm,),
        in_specs=[
            pl.BlockSpec((tm, d), col),
            pl.BlockSpec((tm, d), col),
            pl.BlockSpec((tm, 1), col),
            pl.BlockSpec((tm, 1), col),
        ],
        out_specs=pl.BlockSpec((tm, d), col),
        out_shape=jax.ShapeDtypeStruct((t, d), jnp.float32),
    )(y0, y1, p0, p1)


def _moe_slice(x2, eps, Wg, bg, Wn, bn, W1b, b1, W2b, b2):
    t, d = x2.shape
    e = Wg.shape[-1]
    tmr = 512 if t % 512 == 0 else t
    i0, i1, r0, r1, p0, p1, cnt = _router(x2, Wg, bg, Wn, bn, eps, tmr)

    tm = _TM if t % _TM == 0 else t
    n_pad = 2 * t + e * tm
    n_tiles = n_pad // tm
    pos0, pos1, emap = _pos(cnt, i0, i1, r0, r1, tm, n_tiles)
    pos0f, pos1f = pos0.reshape(t), pos1.reshape(t)

    xs = _dispatch_sc(x2, pos0f, pos1f, n_pad)
    ys = _ffn(xs, W1b, b1, W2b, b2, emap.reshape(n_tiles), tm)
    y0, y1 = _gather_sc(ys, pos0f, pos1f, t)
    return _combine(y0, y1, p0, p1, 1024 if t % 1024 == 0 else t)


def kernel(x, Wg, bg, Wn, bn, W1, b1, W2, b2):
    b, s, d = x.shape
    e = Wg.shape[-1]
    t = b * s
    x2 = x.reshape(t, d)
    eps = jax.random.normal(jax.random.key(42), (b, s, e),
                            dtype=jnp.float32).reshape(t, e)
    W1b, W2b = W1.astype(jnp.bfloat16), W2.astype(jnp.bfloat16)
    out2 = _moe_slice(x2, eps, Wg, bg, Wn, bn, W1b, b1, W2b, b2)
    return out2.reshape(b, s, d)
